# Initial kernel scaffold; baseline (speedup 1.0000x reference)
import jax
import jax.numpy as jnp
from jax.experimental import pallas as pl


def kernel(h, adj, W):
    raise NotImplementedError("write your pallas kernel here")


# speedup vs baseline: 1.0091x; 1.0091x over previous
import functools

import jax
import jax.numpy as jnp
from jax.experimental import pallas as pl
from jax.experimental.pallas import tpu as pltpu


def _hw_body(h_ref, w_ref, hw_ref):
    hw_ref[...] = jnp.dot(
        h_ref[...].astype(jnp.bfloat16),
        w_ref[...].astype(jnp.bfloat16),
        preferred_element_type=jnp.float32,
    ).astype(jnp.bfloat16)


def _av_body(adj_ref, hw_ref, out_ref):
    a = adj_ref[...].astype(jnp.bfloat16)
    out_ref[...] = jnp.dot(a, hw_ref[...], preferred_element_type=jnp.float32)


def kernel(h, adj, W):
    n, in_f = h.shape
    out_f = W.shape[1]

    br_hw = 2000
    hw = pl.pallas_call(
        _hw_body,
        grid=(n // br_hw,),
        in_specs=[
            pl.BlockSpec((br_hw, in_f), lambda i: (i, 0)),
            pl.BlockSpec((in_f, out_f), lambda i: (0, 0)),
        ],
        out_specs=pl.BlockSpec((br_hw, out_f), lambda i: (i, 0)),
        out_shape=jax.ShapeDtypeStruct((n, out_f), jnp.bfloat16),
    )(h, W)

    br = 400
    out = pl.pallas_call(
        _av_body,
        grid=(n // br,),
        in_specs=[
            pl.BlockSpec((br, n), lambda i: (i, 0)),
            pl.BlockSpec((n, out_f), lambda i: (0, 0)),
        ],
        out_specs=pl.BlockSpec((br, out_f), lambda i: (i, 0)),
        out_shape=jax.ShapeDtypeStruct((n, out_f), jnp.float32),
        compiler_params=pltpu.CompilerParams(
            dimension_semantics=("arbitrary",),
        ),
    )(adj, hw)
    return out


# speedup vs baseline: 1.0357x; 1.0264x over previous
import jax
import jax.numpy as jnp
from jax.experimental import pallas as pl
from jax.experimental.pallas import tpu as pltpu


def _body(h_ref, w_ref, adj_ref, out_ref, hw_scr):
    i = pl.program_id(0)

    @pl.when(i == 0)
    def _hw():
        hw_scr[...] = jnp.dot(
            h_ref[...].astype(jnp.bfloat16),
            w_ref[...].astype(jnp.bfloat16),
            preferred_element_type=jnp.float32,
        ).astype(jnp.bfloat16)

    @pl.when(i > 0)
    def _mm():
        out_ref[...] = jnp.dot(
            adj_ref[...].astype(jnp.bfloat16),
            hw_scr[...],
            preferred_element_type=jnp.float32,
        )


def kernel(h, adj, W):
    n, in_f = h.shape
    out_f = W.shape[1]
    br = 400

    def _adj_idx(i):
        return (jnp.maximum(i - 1, 0), 0)

    out = pl.pallas_call(
        _body,
        grid=(n // br + 1,),
        in_specs=[
            pl.BlockSpec((n, in_f), lambda i: (0, 0)),
            pl.BlockSpec((in_f, out_f), lambda i: (0, 0)),
            pl.BlockSpec((br, n), _adj_idx),
        ],
        out_specs=pl.BlockSpec((br, out_f), _adj_idx),
        out_shape=jax.ShapeDtypeStruct((n, out_f), jnp.float32),
        scratch_shapes=[pltpu.VMEM((n, out_f), jnp.bfloat16)],
        compiler_params=pltpu.CompilerParams(
            dimension_semantics=("arbitrary",),
        ),
    )(h, W, adj)
    return out


# speedup vs baseline: 1.0404x; 1.0045x over previous
import jax
import jax.numpy as jnp
from jax.experimental import pallas as pl
from jax.experimental.pallas import tpu as pltpu


def _body(h_ref, w_ref, adj_ref, out_ref, hw_scr):
    i = pl.program_id(0)

    @pl.when(i == 0)
    def _hw():
        hw_scr[...] = jnp.dot(
            h_ref[...].astype(jnp.bfloat16),
            w_ref[...].astype(jnp.bfloat16),
            preferred_element_type=jnp.float32,
        ).astype(jnp.bfloat16)

    @pl.when(i > 0)
    def _mm():
        out_ref[...] = jnp.dot(
            adj_ref[...].astype(jnp.bfloat16),
            hw_scr[...],
            preferred_element_type=jnp.float32,
        )


def kernel(h, adj, W):
    n, in_f = h.shape
    out_f = W.shape[1]
    br = 200

    def _adj_idx(i):
        return (jnp.maximum(i - 1, 0), 0)

    out = pl.pallas_call(
        _body,
        grid=(n // br + 1,),
        in_specs=[
            pl.BlockSpec((n, in_f), lambda i: (0, 0)),
            pl.BlockSpec((in_f, out_f), lambda i: (0, 0)),
            pl.BlockSpec((br, n), _adj_idx),
        ],
        out_specs=pl.BlockSpec((br, out_f), _adj_idx),
        out_shape=jax.ShapeDtypeStruct((n, out_f), jnp.float32),
        scratch_shapes=[pltpu.VMEM((n, out_f), jnp.bfloat16)],
        compiler_params=pltpu.CompilerParams(
            dimension_semantics=("arbitrary",),
        ),
    )(h, W, adj)
    return out
